# baseline (speedup 1.0000x reference)
import functools

import jax
import jax.numpy as jnp
from jax import lax
from jax.experimental import pallas as pl
from jax.experimental.pallas import tpu as pltpu
from jax.experimental.pallas import tpu_sc as plsc

_info = plsc.get_sparse_core_info()
_NC, _NS = _info.num_cores, _info.num_subcores
_NW = _NC * _NS

_V = 512
_D = 768
_B = 64 * 512
_RPW = _V // _NW
_C = 16
_UNROLL = 8

_mesh = plsc.VectorSubcoreMesh(core_axis_name="c", subcore_axis_name="s")


@functools.partial(
    pl.kernel,
    mesh=_mesh,
    compiler_params=pltpu.CompilerParams(needs_layout_passes=False),
    out_type=jax.ShapeDtypeStruct((_B, _D), jnp.float32),
    scratch_types=[
        pltpu.VMEM((_B,), jnp.int32),
        pltpu.VMEM((_B,), jnp.int32),
        pltpu.VMEM((_RPW, _D), jnp.float32),
        pltpu.VMEM((_C, _D), jnp.float32),
        pltpu.VMEM((_C, _D), jnp.float32),
        pltpu.VMEM((_C,), jnp.int32),
        pltpu.VMEM((_C,), jnp.int32),
        pltpu.SemaphoreType.DMA,
    ],
)
def _scatter_kernel(table_hbm, idx_hbm, out_hbm, idx_v, mpack, tpart,
                    repA, repB, pos0, pos1, ssem):
    wid = lax.axis_index("s") * _NC + lax.axis_index("c")
    lo = wid * _RPW
    pltpu.sync_copy(table_hbm.at[pl.ds(lo, _RPW)], tpart)
    pltpu.sync_copy(idx_hbm, idx_v)

    iota = lax.iota(jnp.int32, 16)
    lane15 = iota * 0 + 15
    dn1 = lax.GatherDimensionNumbers(
        offset_dims=(), collapsed_slice_dims=(0,), start_index_map=(0,))

    def bcast_last(x):
        return lax.gather(x, lane15[:, None], dn1, slice_sizes=(1,),
                          mode=lax.GatherScatterMode.PROMISE_IN_BOUNDS)

    def scan_body(i0, offv):
        for u in range(_UNROLL):
            i = i0 * _UNROLL + u
            v = idx_v[pl.ds(i * 16, 16)]
            t = v - lo
            m = (t >= 0) & (t < _RPW)
            pk = ((iota + i * 16) << 4) | (t & (_RPW - 1))
            pfx = plsc.cumsum(m.astype(jnp.int32))
            plsc.store_scatter(mpack, [offv + pfx - 1], pk, mask=m)
            offv = offv + bcast_last(pfx)
        return offv

    nmatch = lax.fori_loop(0, _B // 16 // _UNROLL, scan_body, iota * 0)[0]
    nmv = (nmatch + 15) // 16

    bins = idx_v
    reps = (repA, repB)
    poss = (pos0, pos1)

    def drain():
        pltpu.make_async_copy(out_hbm.at[pl.ds(0, _C)], repA, ssem).wait()

    prev_nsub = jnp.int32(0)
    for r in range(_RPW):
        rep = reps[r % 2]

        def bin_body(q, boffv, r=r):
            pk = mpack[pl.ds(q * 16, 16)]
            m = ((q * 16 + iota) < nmatch) & ((pk & (_RPW - 1)) == r)
            pfx = plsc.cumsum(m.astype(jnp.int32))
            plsc.store_scatter(bins, [boffv + pfx - 1], pk >> 4, mask=m)
            return boffv + bcast_last(pfx)

        k_r = lax.fori_loop(0, nmv, bin_body, iota * 0)[0]

        def rep_body(cc, _, r=r, rep=rep):
            for w in range(2):
                val = tpart[r, pl.ds((cc * 2 + w) * 16, 16)]
                for j in range(_C):
                    rep[j, pl.ds((cc * 2 + w) * 16, 16)] = val
            return 0

        lax.fori_loop(0, _D // 32, rep_body, 0)

        @pl.when(prev_nsub >= 1)
        def _():
            drain()

        @pl.when(prev_nsub >= 2)
        def _():
            drain()

        nsub = (k_r + (_C - 1)) // _C

        def pair_body(uu, _, rep=rep, k_r=k_r, nsub=nsub):
            for h in range(2):
                u = uu * 2 + h

                @pl.when(u * 1 < nsub)
                def _(u=u, h=h, rep=rep, k_r=k_r):
                    @pl.when(u >= 2)
                    def _():
                        drain()

                    rem = jnp.minimum(k_r - u * _C, _C)
                    src = jnp.minimum(iota, rem - 1)
                    pv = bins[pl.ds(u * _C, 16)]
                    pv = lax.gather(
                        pv, src[:, None],
                        lax.GatherDimensionNumbers(
                            offset_dims=(), collapsed_slice_dims=(0,),
                            start_index_map=(0,)),
                        slice_sizes=(1,),
                        mode=lax.GatherScatterMode.PROMISE_IN_BOUNDS)
                    poss[h][...] = pv
                    pltpu.async_copy(rep, out_hbm.at[poss[h]], ssem)
            return 0

        lax.fori_loop(0, (nsub + 1) // 2, pair_body, 0)
        prev_nsub = jnp.minimum(nsub, 2)

    @pl.when(prev_nsub >= 1)
    def _():
        drain()

    @pl.when(prev_nsub >= 2)
    def _():
        drain()


@jax.jit
def kernel(position_ids, table):
    b, s = position_ids.shape
    idx = position_ids.reshape(-1).astype(jnp.int32)
    out = _scatter_kernel(table, idx)
    return out.reshape(b, s, _D)


# speedup vs baseline: 1.1332x; 1.1332x over previous
import functools

import jax
import jax.numpy as jnp
from jax import lax
from jax.experimental import pallas as pl
from jax.experimental.pallas import tpu as pltpu
from jax.experimental.pallas import tpu_sc as plsc

_info = plsc.get_sparse_core_info()
_NC, _NS = _info.num_cores, _info.num_subcores
_NW = _NC * _NS

_V = 512
_D = 768
_B = 64 * 512
_RPW = _V // _NW
_C = 16
_UNROLL = 8

_mesh = plsc.VectorSubcoreMesh(core_axis_name="c", subcore_axis_name="s")


@functools.partial(
    pl.kernel,
    mesh=_mesh,
    compiler_params=pltpu.CompilerParams(needs_layout_passes=False),
    out_type=jax.ShapeDtypeStruct((_B, _D), jnp.float32),
    scratch_types=[
        pltpu.VMEM((_B + 16,), jnp.int32),
        pltpu.VMEM((_B,), jnp.int32),
        pltpu.VMEM((_RPW, _D), jnp.float32),
        pltpu.VMEM((_C, _D), jnp.float32),
        pltpu.VMEM((_C, _D), jnp.float32),
        pltpu.VMEM((_C,), jnp.int32),
        pltpu.VMEM((_C,), jnp.int32),
        pltpu.VMEM((_C,), jnp.int32),
        pltpu.VMEM((_C,), jnp.int32),
        pltpu.SemaphoreType.DMA,
        pltpu.SemaphoreType.DMA,
        pltpu.SemaphoreType.DMA,
    ],
)
def _scatter_kernel(table_hbm, idx_hbm, out_hbm, idx_v, mpack, tpart,
                    repA, repB, pos0, pos1, pos2, pos3, ssem, rsem, isem):
    wid = lax.axis_index("s") * _NC + lax.axis_index("c")
    lo = wid * _RPW
    pltpu.sync_copy(table_hbm.at[pl.ds(lo, _RPW)], tpart)
    iota = lax.iota(jnp.int32, 16)

    _ICH = _B // 8

    def idx_copy(c):
        return pltpu.async_copy(
            idx_hbm.at[pl.ds(c * _ICH, _ICH)],
            idx_v.at[pl.ds(c * _ICH, _ICH)], isem)

    def scan_body(i0, off):
        for u in range(_UNROLL):
            i = i0 * _UNROLL + u
            v = idx_v[pl.ds(i * 16, 16)]
            m = (v >> 4) == wid
            pk = ((iota + i * 16) << 4) | (v & (_RPW - 1))
            plsc.store_compressed(mpack.at[pl.ds(off, 16)], pk, mask=m)
            off = off + plsc.all_reduce_population_count(m)[0]
        return off

    cps = [idx_copy(0)]
    nmatch = 0
    for c in range(8):
        cps[c].wait()
        if c + 1 < 8:
            cps.append(idx_copy(c + 1))
        base_i = c * (_ICH // 16 // _UNROLL)
        nmatch = lax.fori_loop(base_i, base_i + _ICH // 16 // _UNROLL,
                               scan_body, nmatch)
    nmv = (nmatch + 15) // 16

    bins = idx_v
    reps = (repA, repB)
    poss = (pos0, pos1, pos2, pos3)

    def drain():
        pltpu.make_async_copy(out_hbm.at[pl.ds(0, _C)], repA, ssem).wait()

    prev_nsub = jnp.int32(0)
    for r in range(_RPW):
        rep = reps[r % 2]

        def bin_body(q, boff, r=r):
            pk = mpack[pl.ds(q * 16, 16)]
            m = ((q * 16 + iota) < nmatch) & ((pk & (_RPW - 1)) == r)
            plsc.store_compressed(bins.at[pl.ds(boff, 16)], pk >> 4, mask=m)
            return boff + plsc.all_reduce_population_count(m)[0]

        k_r = lax.fori_loop(0, nmv, bin_body, 0)

        @pl.when(k_r > 0)
        def _():
            b0 = bins[pl.ds(0, 16)]
            pad = lax.gather(
                b0, (iota * 0)[:, None],
                lax.GatherDimensionNumbers(
                    offset_dims=(), collapsed_slice_dims=(0,),
                    start_index_map=(0,)),
                slice_sizes=(1,),
                mode=lax.GatherScatterMode.PROMISE_IN_BOUNDS)
            bins[pl.ds(k_r, 16)] = pad

        def rep_body(cc, _, r=r, rep=rep):
            for w in range(2):
                val = tpart[r, pl.ds((cc * 2 + w) * 16, 16)]
                for j in range(_C):
                    rep[j, pl.ds((cc * 2 + w) * 16, 16)] = val
            return 0

        lax.fori_loop(0, _D // 32, rep_body, 0)

        for k in range(1, 5):
            @pl.when(prev_nsub >= k)
            def _():
                drain()

        nsub = (k_r + (_C - 1)) // _C

        def quad_body(uu, _, rep=rep, nsub=nsub):
            for h in range(4):
                u = uu * 4 + h

                @pl.when(u * 1 < nsub)
                def _(u=u, h=h, rep=rep):
                    @pl.when(u >= 4)
                    def _():
                        drain()

                    poss[h][...] = bins[pl.ds(u * _C, 16)]
                    pltpu.async_copy(rep, out_hbm.at[poss[h]], ssem)
            return 0

        lax.fori_loop(0, (nsub + 3) // 4, quad_body, 0)
        prev_nsub = jnp.minimum(nsub, 4)

    for k in range(1, 5):
        @pl.when(prev_nsub >= k)
        def _():
            drain()


@jax.jit
def kernel(position_ids, table):
    b, s = position_ids.shape
    idx = position_ids.reshape(-1).astype(jnp.int32)
    out = _scatter_kernel(table, idx)
    return out.reshape(b, s, _D)
